# Initial kernel scaffold; baseline (speedup 1.0000x reference)
import jax
import jax.numpy as jnp
from jax.experimental import pallas as pl


def kernel(input_ids, token_table, position_table):
    raise NotImplementedError("write your pallas kernel here")


# speedup vs baseline: 4.2568x; 4.2568x over previous
import functools

import jax
import jax.numpy as jnp
from jax import lax
from jax.experimental import pallas as pl
from jax.experimental.pallas import tpu as pltpu
from jax.experimental.pallas import tpu_sc as plsc


def _sc_embed(ids_flat, token_table, position_table, *, B, L, D):
    NC, NS = 2, 16
    NW = NC * NS
    BPW = B // NW
    n_rows = BPW * L

    mesh = plsc.VectorSubcoreMesh(core_axis_name="c", subcore_axis_name="s")

    @functools.partial(
        pl.kernel,
        mesh=mesh,
        out_type=jax.ShapeDtypeStruct((B * L, D), jnp.float32),
        scratch_types=[
            pltpu.VMEM((n_rows,), jnp.int32),
            pltpu.VMEM((L, D), jnp.float32),
            pltpu.VMEM((L, D), jnp.float32),
            pltpu.SemaphoreType.DMA,
        ],
    )
    def k(ids_hbm, tbl_hbm, pos_hbm, out_hbm, idx_v, pos_v, buf, sem):
        wid = lax.axis_index("s") * NC + lax.axis_index("c")
        base = wid * n_rows
        pltpu.sync_copy(ids_hbm.at[pl.ds(base, n_rows)], idx_v)
        pltpu.sync_copy(pos_hbm.at[pl.ds(0, L)], pos_v)

        def chunk_body(c, carry):
            off = c * L
            cp1 = pltpu.async_copy(
                tbl_hbm.at[idx_v.at[pl.ds(off, 128)]], buf.at[pl.ds(0, 128)], sem)
            cp2 = pltpu.async_copy(
                tbl_hbm.at[idx_v.at[pl.ds(off + 128, L - 128)]],
                buf.at[pl.ds(128, L - 128)], sem)
            cp1.wait()
            cp2.wait()

            def add_row(l, carry2):
                for j in range(D // 16):
                    sl = pl.ds(j * 16, 16)
                    buf[l, sl] = buf[l, sl] + pos_v[l, sl]
                return carry2

            lax.fori_loop(0, L, add_row, 0)
            pltpu.sync_copy(buf, out_hbm.at[pl.ds(base + off, L)])
            return carry

        lax.fori_loop(0, BPW, chunk_body, 0)

    return k(ids_flat, token_table, position_table)


def kernel(input_ids, token_table, position_table):
    B, L = input_ids.shape
    _, D = token_table.shape
    ids_flat = input_ids.reshape(B * L).astype(jnp.int32)
    out = _sc_embed(ids_flat, token_table.astype(jnp.float32),
                    position_table.astype(jnp.float32), B=B, L=L, D=D)
    return out.reshape(B, L, D)


# speedup vs baseline: 7.2681x; 1.7074x over previous
import functools

import jax
import jax.numpy as jnp
from jax import lax
from jax.experimental import pallas as pl
from jax.experimental.pallas import tpu as pltpu
from jax.experimental.pallas import tpu_sc as plsc


def _sc_embed(ids_flat, token_table, position_table, *, B, L, D):
    NC, NS = 2, 16
    NW = NC * NS
    BPW = B // NW
    n_rows = BPW * L
    NBUF = 3

    mesh = plsc.VectorSubcoreMesh(core_axis_name="c", subcore_axis_name="s")

    @functools.partial(
        pl.kernel,
        mesh=mesh,
        out_type=jax.ShapeDtypeStruct((B * L, D), jnp.float32),
        scratch_types=[
            pltpu.VMEM((n_rows,), jnp.int32),
            pltpu.VMEM((L, D), jnp.float32),
        ] + [pltpu.VMEM((L, D), jnp.float32) for _ in range(NBUF)]
          + [pltpu.SemaphoreType.DMA for _ in range(2 * NBUF)],
    )
    def k(ids_hbm, tbl_hbm, pos_hbm, out_hbm, idx_v, pos_v, *bufs_and_sems):
        bufs = bufs_and_sems[:NBUF]
        gsem = bufs_and_sems[NBUF:2 * NBUF]
        ssem = bufs_and_sems[2 * NBUF:]

        wid = lax.axis_index("s") * NC + lax.axis_index("c")
        base = wid * n_rows
        pltpu.sync_copy(ids_hbm.at[pl.ds(base, n_rows)], idx_v)
        pltpu.sync_copy(pos_hbm.at[pl.ds(0, L)], pos_v)

        def issue_gather(c):
            b = c % NBUF
            off = c * L
            cp1 = pltpu.async_copy(
                tbl_hbm.at[idx_v.at[pl.ds(off, 128)]],
                bufs[b].at[pl.ds(0, 128)], gsem[b])
            cp2 = pltpu.async_copy(
                tbl_hbm.at[idx_v.at[pl.ds(off + 128, L - 128)]],
                bufs[b].at[pl.ds(128, L - 128)], gsem[b])
            return (cp1, cp2)

        gathers = {}
        scatters = {}
        gathers[0] = issue_gather(0)
        gathers[1] = issue_gather(1)

        for c in range(BPW):
            b = c % NBUF
            gathers.pop(c)[0].wait()
            gathers[c] = None
            pltpu.make_async_copy(
                tbl_hbm.at[idx_v.at[pl.ds(c * L + 128, L - 128)]],
                bufs[b].at[pl.ds(128, L - 128)], gsem[b]).wait()

            buf = bufs[b]

            def add_row(l, carry, buf=buf):
                for j in range(D // 16):
                    sl = pl.ds(j * 16, 16)
                    buf[l, sl] = buf[l, sl] + pos_v[l, sl]
                return carry

            lax.fori_loop(0, L, add_row, 0)

            scatters[c] = pltpu.async_copy(
                buf, out_hbm.at[pl.ds(base + c * L, L)], ssem[b])

            if c + 2 < BPW:
                if c >= 1:
                    scatters.pop(c - 1).wait()
                gathers[c + 2] = issue_gather(c + 2)

        for c in sorted(scatters):
            scatters[c].wait()

    return k(ids_flat, token_table, position_table)


def kernel(input_ids, token_table, position_table):
    B, L = input_ids.shape
    _, D = token_table.shape
    ids_flat = input_ids.reshape(B * L).astype(jnp.int32)
    out = _sc_embed(ids_flat, token_table.astype(jnp.float32),
                    position_table.astype(jnp.float32), B=B, L=L, D=D)
    return out.reshape(B, L, D)
